# baseline (speedup 1.0000x reference)
import functools

import jax
import jax.numpy as jnp
from jax import lax
from jax.experimental import pallas as pl
from jax.experimental.pallas import tpu as pltpu
from jax.experimental.pallas import tpu_sc as plsc

E = 64
K = 8
T = 16384
NW = 32
TPW = T // NW


def _scores_block(x_ref, w_ref, b_ref, scores_ref):
    x = x_ref[...]
    w = w_ref[...]
    lt = jax.lax.dot_general(
        w, x, (((1,), (1,)), ((), ())),
        preferred_element_type=jnp.float32)
    lt = lt + b_ref[...][:, None]

    m = jnp.max(lt, axis=0, keepdims=True)
    e = jnp.exp(lt - m)
    scores_ref[...] = (e / jnp.sum(e, axis=0, keepdims=True)).T


def _tc_scores(flat, weight, bias):
    Hn = flat.shape[1]
    TB = 1024
    return pl.pallas_call(
        _scores_block,
        grid=(T // TB,),
        in_specs=[
            pl.BlockSpec((TB, Hn), lambda i: (i, 0)),
            pl.BlockSpec((E, Hn), lambda i: (0, 0)),
            pl.BlockSpec((E,), lambda i: (0,)),
        ],
        out_specs=pl.BlockSpec((TB, E), lambda i: (i, 0)),
        out_shape=jax.ShapeDtypeStruct((T, E), jnp.float32),
        compiler_params=pltpu.CompilerParams(
            dimension_semantics=("arbitrary",),
        ),
    )(flat, weight, bias)


def _sc_topk_body(scores_hbm, out_hbm, sbuf, obuf):
    wid = lax.axis_index("s") * 2 + lax.axis_index("c")
    base = pl.multiple_of(wid * TPW, TPW)
    obase = pl.multiple_of(wid * (TPW // 2), TPW // 2)
    pltpu.sync_copy(scores_hbm.at[pl.ds(base, TPW), :], sbuf)

    iota = lax.iota(jnp.int32, 16)
    low8 = iota < 8

    def perm(x, pidx):
        return lax.gather(
            x, pidx[:, None],
            lax.GatherDimensionNumbers(
                offset_dims=(), collapsed_slice_dims=(0,),
                start_index_map=(0,)),
            slice_sizes=(1,),
            mode=lax.GatherScatterMode.PROMISE_IN_BOUNDS)

    def merge(ka, va, kb, vb):
        kc = jnp.where(low8, ka, lax.rev(kb, (0,)))
        vc = jnp.where(low8, va, lax.rev(vb, (0,)))
        for d in (8, 4, 2, 1):
            pidx = iota ^ d
            hi = (iota & d) != 0
            pk = perm(kc, pidx)
            pv = perm(vc, pidx)
            takep = jnp.logical_xor(pk > kc, hi)
            kc = jnp.where(takep, pk, kc)
            vc = jnp.where(takep, pv, vc)
        return kc, vc

    def leaf_sorts(tok):
        ks, vs = [], []
        for j in range(4):
            s = sbuf[tok, pl.ds(16 * j, 16)]
            k_, v_ = plsc.sort_key_val(s, iota + 16 * j, descending=True)
            ks.append(k_)
            vs.append(v_)
        return ks, vs

    def top16(ks, vs):
        k01, v01 = merge(ks[0], vs[0], ks[1], vs[1])
        k23, v23 = merge(ks[2], vs[2], ks[3], vs[3])
        _, vf = merge(k01, v01, k23, v23)
        return vf

    @plsc.parallel_loop(0, TPW // 2, unroll=4)
    def pair_body(p):
        t = 2 * p
        kst, vst = leaf_sorts(t)
        ksu, vsu = leaf_sorts(t + 1)
        vt = top16(kst, vst)
        vu = top16(ksu, vsu)
        vu_rot = perm(vu, iota ^ 8)
        obuf[p, :] = jnp.where(low8, vt, vu_rot)

    pltpu.sync_copy(obuf, out_hbm.at[pl.ds(obase, TPW // 2), :])


_sc_topk = functools.partial(
    pl.kernel,
    out_type=jax.ShapeDtypeStruct((T // 2, 16), jnp.int32),
    mesh=plsc.VectorSubcoreMesh(core_axis_name="c", subcore_axis_name="s"),
    scratch_types=[
        pltpu.VMEM((TPW, E), jnp.float32),
        pltpu.VMEM((TPW // 2, 16), jnp.int32),
    ],
    compiler_params=pltpu.CompilerParams(needs_layout_passes=False),
)(_sc_topk_body)


def kernel(hidden_states, weight, bias):
    Bn, Sn, Hn = hidden_states.shape
    flat = hidden_states.reshape(Bn * Sn, Hn)
    scores = _tc_scores(flat, weight, bias)
    idx = _sc_topk(scores)
    return (scores, idx.reshape(T, K))


# speedup vs baseline: 1.0758x; 1.0758x over previous
import functools

import jax
import jax.numpy as jnp
from jax import lax
from jax.experimental import pallas as pl
from jax.experimental.pallas import tpu as pltpu
from jax.experimental.pallas import tpu_sc as plsc

E = 64
K = 8
T = 16384
NW = 32
TPW = T // NW


def _scores_block(x_ref, w_ref, b_ref, scores_ref):
    x = x_ref[...]
    w = w_ref[...]
    lt = jax.lax.dot_general(
        w, x, (((1,), (1,)), ((), ())),
        preferred_element_type=jnp.float32)
    lt = lt + b_ref[...][:, None]

    m = jnp.max(lt, axis=0, keepdims=True)
    e = jnp.exp(lt - m)
    scores_ref[...] = (e / jnp.sum(e, axis=0, keepdims=True)).T


def _tc_scores(flat, weight, bias):
    Hn = flat.shape[1]
    TB = 1024
    return pl.pallas_call(
        _scores_block,
        grid=(T // TB,),
        in_specs=[
            pl.BlockSpec((TB, Hn), lambda i: (i, 0)),
            pl.BlockSpec((E, Hn), lambda i: (0, 0)),
            pl.BlockSpec((E,), lambda i: (0,)),
        ],
        out_specs=pl.BlockSpec((TB, E), lambda i: (i, 0)),
        out_shape=jax.ShapeDtypeStruct((T, E), jnp.float32),
        compiler_params=pltpu.CompilerParams(
            dimension_semantics=("arbitrary",),
        ),
    )(flat, weight, bias)


def _sc_topk_body(scores_hbm, out_hbm, sbuf, obuf):
    wid = lax.axis_index("s") * 2 + lax.axis_index("c")
    base = pl.multiple_of(wid * TPW, TPW)
    obase = pl.multiple_of(wid * (TPW // 2), TPW // 2)
    pltpu.sync_copy(scores_hbm.at[pl.ds(base, TPW), :], sbuf)

    iota = lax.iota(jnp.int32, 16)
    low8 = iota < 8

    def perm(x, pidx):
        return lax.gather(
            x, pidx[:, None],
            lax.GatherDimensionNumbers(
                offset_dims=(), collapsed_slice_dims=(0,),
                start_index_map=(0,)),
            slice_sizes=(1,),
            mode=lax.GatherScatterMode.PROMISE_IN_BOUNDS)

    def merge(ka, va, kb, vb):
        kc = jnp.where(low8, ka, lax.rev(kb, (0,)))
        vc = jnp.where(low8, va, lax.rev(vb, (0,)))
        return plsc.sort_key_val(kc, vc, descending=True)

    def leaf_sorts(tok):
        ks, vs = [], []
        for j in range(4):
            s = sbuf[tok, pl.ds(16 * j, 16)]
            k_, v_ = plsc.sort_key_val(s, iota + 16 * j, descending=True)
            ks.append(k_)
            vs.append(v_)
        return ks, vs

    def top16(ks, vs):
        k01, v01 = merge(ks[0], vs[0], ks[1], vs[1])
        k23, v23 = merge(ks[2], vs[2], ks[3], vs[3])
        _, vf = merge(k01, v01, k23, v23)
        return vf

    @plsc.parallel_loop(0, TPW // 2, unroll=4)
    def pair_body(p):
        t = 2 * p
        kst, vst = leaf_sorts(t)
        ksu, vsu = leaf_sorts(t + 1)
        vt = top16(kst, vst)
        vu = top16(ksu, vsu)
        vu_rot = perm(vu, iota ^ 8)
        obuf[p, :] = jnp.where(low8, vt, vu_rot)

    pltpu.sync_copy(obuf, out_hbm.at[pl.ds(obase, TPW // 2), :])


_sc_topk = functools.partial(
    pl.kernel,
    out_type=jax.ShapeDtypeStruct((T // 2, 16), jnp.int32),
    mesh=plsc.VectorSubcoreMesh(core_axis_name="c", subcore_axis_name="s"),
    scratch_types=[
        pltpu.VMEM((TPW, E), jnp.float32),
        pltpu.VMEM((TPW // 2, 16), jnp.int32),
    ],
    compiler_params=pltpu.CompilerParams(needs_layout_passes=False),
)(_sc_topk_body)


def kernel(hidden_states, weight, bias):
    Bn, Sn, Hn = hidden_states.shape
    flat = hidden_states.reshape(Bn * Sn, Hn)
    scores = _tc_scores(flat, weight, bias)
    idx = _sc_topk(scores)
    return (scores, idx.reshape(T, K))
